# Initial kernel scaffold; baseline (speedup 1.0000x reference)
import jax
import jax.numpy as jnp
from jax.experimental import pallas as pl


def kernel(x, importance, Wc, WQ, WK, WV):
    raise NotImplementedError("write your pallas kernel here")


# speedup vs baseline: 1.0662x; 1.0662x over previous
import jax
import jax.numpy as jnp
from jax.experimental import pallas as pl
from jax.experimental.pallas import tpu as pltpu

_B, _S, _D = 4, 2048, 2048
_N = 64
_NR = 4
_KS = (16, 8, 8, 8)
_SBLK = 512
_NSB = _S // _SBLK


def _router_kernel(x_ref, imp_ref, w_ref,
                   cw_ref, ci_ref, qw_ref, qi_ref, kw_ref, ki_ref,
                   vw_ref, vi_ref, cm_ref, qm_ref, km_ref, vm_ref,
                   acc_ref):
    s = pl.program_id(1)

    @pl.when(s == 0)
    def _zero():
        acc_ref[...] = jnp.zeros_like(acc_ref)

    x = x_ref[0].astype(jnp.bfloat16)
    imp = imp_ref[0, 0].astype(jnp.bfloat16).astype(jnp.float32)
    logits = jax.lax.dot_general(
        x, w_ref[...].astype(jnp.bfloat16), (((1,), (0,)), ((), ())),
        preferred_element_type=jnp.float32)

    for r in range(_NR):
        lg = logits[:, r * _N:(r + 1) * _N]
        m = jnp.max(lg, axis=-1, keepdims=True)
        e = jnp.exp(lg - m)
        esum = jnp.sum(e, axis=-1, keepdims=True)
        p = (e / esum).astype(jnp.bfloat16).astype(jnp.float32)
        acc_ref[r:r + 1, :] += jnp.sum(p * imp, axis=0, keepdims=True)

    @pl.when(s == _NSB - 1)
    def _epilogue():
        iota = jax.lax.broadcasted_iota(jnp.int32, (1, _N), 1)
        w_refs = (cw_ref, qw_ref, kw_ref, vw_ref)
        i_refs = (ci_ref, qi_ref, ki_ref, vi_ref)
        m_refs = (cm_ref, qm_ref, km_ref, vm_ref)
        for r in range(_NR):
            k = _KS[r]
            kiota = jax.lax.broadcasted_iota(jnp.int32, (1, k), 1)
            w = acc_ref[r:r + 1, :]
            wn = w / (jnp.sum(w) + 1e-8)
            vals = wn
            mask = jnp.zeros((1, _N), jnp.float32)
            wvec = jnp.zeros((1, k), jnp.float32)
            ivec = jnp.zeros((1, k), jnp.int32)
            tsum = jnp.float32(0.0)
            for j in range(k):
                mx = jnp.max(vals)
                ix = jnp.min(jnp.where(vals == mx, iota, _N))
                tsum = tsum + mx
                wvec = jnp.where(kiota == j, mx, wvec)
                ivec = jnp.where(kiota == j, ix, ivec)
                hit = iota == ix
                mask = jnp.where(hit, 1.0, mask)
                vals = jnp.where(hit, -1.0, vals)
            inv = 1.0 / (tsum + 1e-8)
            w_refs[r][...] = (wvec * inv).reshape(1, 1, k)
            i_refs[r][...] = ivec.reshape(1, 1, k)
            m_refs[r][...] = mask.reshape(1, 1, _N)


def kernel(x, importance, Wc, WQ, WK, WV):
    w_all = jnp.concatenate([Wc, WQ, WK, WV], axis=1)
    imp4 = importance.reshape(_B, _NSB, _SBLK, 1)

    def out(shape, dtype=jnp.float32):
        return jax.ShapeDtypeStruct(shape, dtype)

    out_shapes = (
        out((_B, 1, _KS[0])), out((_B, 1, _KS[0]), jnp.int32),
        out((_B, 1, _KS[1])), out((_B, 1, _KS[1]), jnp.int32),
        out((_B, 1, _KS[2])), out((_B, 1, _KS[2]), jnp.int32),
        out((_B, 1, _KS[3])), out((_B, 1, _KS[3]), jnp.int32),
        out((_B, 1, _N)), out((_B, 1, _N)), out((_B, 1, _N)), out((_B, 1, _N)),
    )

    def b_map(b, s):
        return (b, 0, 0)

    out_specs = tuple(
        pl.BlockSpec((1, 1, sh.shape[2]), b_map) for sh in out_shapes)

    res = pl.pallas_call(
        _router_kernel,
        grid=(_B, _NSB),
        in_specs=[
            pl.BlockSpec((1, _SBLK, _D), lambda b, s: (b, s, 0)),
            pl.BlockSpec((1, 1, _SBLK, 1), lambda b, s: (b, s, 0, 0)),
            pl.BlockSpec((_D, _NR * _N), lambda b, s: (0, 0)),
        ],
        out_specs=out_specs,
        out_shape=out_shapes,
        scratch_shapes=[pltpu.VMEM((_NR, _N), jnp.float32)],
        compiler_params=pltpu.CompilerParams(
            dimension_semantics=("arbitrary", "arbitrary")),
    )(x, imp4, w_all)

    (cw, ci, qw, qi, kw, ki, vw, vi, cm, qm, km, vm) = res
    sq = lambda a: a.reshape(a.shape[0], a.shape[2])
    return (sq(cw), sq(ci), sq(qw), sq(qi), sq(kw), sq(ki), sq(vw), sq(vi),
            sq(cm), sq(qm), sq(km), sq(vm))


# speedup vs baseline: 1.4561x; 1.3657x over previous
import jax
import jax.numpy as jnp
from jax.experimental import pallas as pl
from jax.experimental.pallas import tpu as pltpu

_B, _S, _D = 4, 2048, 2048
_N = 64
_NR = 4
_KS = (16, 8, 8, 8)
_SBLK = 512
_NSB = _S // _SBLK


def _router_kernel(x_ref, imp_ref, w_ref,
                   cw_ref, ci_ref, qw_ref, qi_ref, kw_ref, ki_ref,
                   vw_ref, vi_ref, cm_ref, qm_ref, km_ref, vm_ref,
                   acc_ref):
    s = pl.program_id(1)

    @pl.when(s == 0)
    def _zero():
        acc_ref[...] = jnp.zeros_like(acc_ref)

    x = x_ref[0].astype(jnp.bfloat16)
    imp = imp_ref[0, 0].astype(jnp.bfloat16)
    logits = jax.lax.dot_general(
        x, w_ref[...].astype(jnp.bfloat16), (((1,), (0,)), ((), ())),
        preferred_element_type=jnp.float32)

    for r in range(_NR):
        lg = logits[:, r * _N:(r + 1) * _N]
        m = jnp.max(lg, axis=-1, keepdims=True)
        e = jnp.exp(lg - m)
        esum = jnp.sum(e, axis=-1, keepdims=True)
        p16 = (e / esum).astype(jnp.bfloat16)
        contrib = jax.lax.dot_general(
            imp, p16, (((1,), (0,)), ((), ())),
            preferred_element_type=jnp.float32)
        acc_ref[r:r + 1, :] += contrib

    @pl.when(s == _NSB - 1)
    def _epilogue():
        iota = jax.lax.broadcasted_iota(jnp.int32, (1, _N), 1)
        w_refs = (cw_ref, qw_ref, kw_ref, vw_ref)
        i_refs = (ci_ref, qi_ref, ki_ref, vi_ref)
        m_refs = (cm_ref, qm_ref, km_ref, vm_ref)
        for r in range(_NR):
            k = _KS[r]
            kiota = jax.lax.broadcasted_iota(jnp.int32, (1, k), 1)
            w = acc_ref[r:r + 1, :]
            wn = w / (jnp.sum(w, axis=-1, keepdims=True) + 1e-8)
            vals = wn
            mask = jnp.zeros((1, _N), jnp.float32)
            wvec = jnp.zeros((1, k), jnp.float32)
            ivec = jnp.zeros((1, k), jnp.int32)
            tsum = jnp.zeros((1, 1), jnp.float32)
            for j in range(k):
                mx = jnp.max(vals, axis=-1, keepdims=True)
                ixv = jnp.min(jnp.where(vals == mx, iota, _N),
                              axis=-1, keepdims=True)
                tsum = tsum + mx
                wvec = jnp.where(kiota == j, mx, wvec)
                ivec = jnp.where(kiota == j, ixv, ivec)
                hit = iota == ixv
                mask = jnp.where(hit, 1.0, mask)
                vals = jnp.where(hit, -1.0, vals)
            inv = 1.0 / (tsum + 1e-8)
            w_refs[r][...] = (wvec * inv).reshape(1, 1, k)
            i_refs[r][...] = ivec.reshape(1, 1, k)
            m_refs[r][...] = mask.reshape(1, 1, _N)


def kernel(x, importance, Wc, WQ, WK, WV):
    w_all = jnp.concatenate([Wc, WQ, WK, WV], axis=1)
    imp4 = importance.reshape(_B, _NSB, 1, _SBLK)

    def out(shape, dtype=jnp.float32):
        return jax.ShapeDtypeStruct(shape, dtype)

    out_shapes = (
        out((_B, 1, _KS[0])), out((_B, 1, _KS[0]), jnp.int32),
        out((_B, 1, _KS[1])), out((_B, 1, _KS[1]), jnp.int32),
        out((_B, 1, _KS[2])), out((_B, 1, _KS[2]), jnp.int32),
        out((_B, 1, _KS[3])), out((_B, 1, _KS[3]), jnp.int32),
        out((_B, 1, _N)), out((_B, 1, _N)), out((_B, 1, _N)), out((_B, 1, _N)),
    )

    def b_map(b, s):
        return (b, 0, 0)

    out_specs = tuple(
        pl.BlockSpec((1, 1, sh.shape[2]), b_map) for sh in out_shapes)

    res = pl.pallas_call(
        _router_kernel,
        grid=(_B, _NSB),
        in_specs=[
            pl.BlockSpec((1, _SBLK, _D), lambda b, s: (b, s, 0)),
            pl.BlockSpec((1, 1, 1, _SBLK), lambda b, s: (b, s, 0, 0)),
            pl.BlockSpec((_D, _NR * _N), lambda b, s: (0, 0)),
        ],
        out_specs=out_specs,
        out_shape=out_shapes,
        scratch_shapes=[pltpu.VMEM((_NR, _N), jnp.float32)],
        compiler_params=pltpu.CompilerParams(
            dimension_semantics=("arbitrary", "arbitrary")),
    )(x, imp4, w_all)

    (cw, ci, qw, qi, kw, ki, vw, vi, cm, qm, km, vm) = res
    sq = lambda a: a.reshape(a.shape[0], a.shape[2])
    return (sq(cw), sq(ci), sq(qw), sq(qi), sq(kw), sq(ki), sq(vw), sq(vi),
            sq(cm), sq(qm), sq(km), sq(vm))


# speedup vs baseline: 1.9862x; 1.3641x over previous
import jax
import jax.numpy as jnp
from jax.experimental import pallas as pl
from jax.experimental.pallas import tpu as pltpu

_B, _S, _D = 4, 2048, 2048
_N = 64
_NR = 4
_KS = (16, 8, 8, 8)
_SBLK = 512
_NSB = _S // _SBLK


def _router_kernel(x_ref, imp_ref, w_ref,
                   cw_ref, ci_ref, qw_ref, qi_ref, kw_ref, ki_ref,
                   vw_ref, vi_ref, cm_ref, qm_ref, km_ref, vm_ref,
                   acc_ref):
    s = pl.program_id(1)

    @pl.when(s == 0)
    def _zero():
        acc_ref[...] = jnp.zeros_like(acc_ref)

    x = x_ref[0].astype(jnp.bfloat16)
    imp = imp_ref[0, 0].astype(jnp.bfloat16)
    logits = jax.lax.dot_general(
        x, w_ref[...].astype(jnp.bfloat16), (((1,), (0,)), ((), ())),
        preferred_element_type=jnp.float32)
    lt = logits.T

    for r in range(_NR):
        lg = lt[r * _N:(r + 1) * _N, :]
        m = jnp.max(lg, axis=0, keepdims=True)
        e = jnp.exp(lg - m)
        esum = jnp.sum(e, axis=0, keepdims=True)
        p16 = (e / esum).astype(jnp.bfloat16)
        contrib = jax.lax.dot_general(
            p16, imp, (((1,), (0,)), ((), ())),
            preferred_element_type=jnp.float32)
        acc_ref[r * _N:(r + 1) * _N, :] += contrib

    @pl.when(s == _NSB - 1)
    def _epilogue():
        iota = jax.lax.broadcasted_iota(jnp.int32, (1, _N), 1)
        w_refs = (cw_ref, qw_ref, kw_ref, vw_ref)
        i_refs = (ci_ref, qi_ref, ki_ref, vi_ref)
        m_refs = (cm_ref, qm_ref, km_ref, vm_ref)
        for r in range(_NR):
            k = _KS[r]
            kiota = jax.lax.broadcasted_iota(jnp.int32, (1, k), 1)
            w = acc_ref[r * _N:(r + 1) * _N, :].T
            wn = w / (jnp.sum(w, axis=-1, keepdims=True) + 1e-8)
            vals = wn
            mask = jnp.zeros((1, _N), jnp.float32)
            wvec = jnp.zeros((1, k), jnp.float32)
            ivec = jnp.zeros((1, k), jnp.int32)
            tsum = jnp.zeros((1, 1), jnp.float32)
            for j in range(k):
                mx = jnp.max(vals, axis=-1, keepdims=True)
                ixv = jnp.min(jnp.where(vals == mx, iota, _N),
                              axis=-1, keepdims=True)
                tsum = tsum + mx
                wvec = jnp.where(kiota == j, mx, wvec)
                ivec = jnp.where(kiota == j, ixv, ivec)
                hit = iota == ixv
                mask = jnp.where(hit, 1.0, mask)
                vals = jnp.where(hit, -1.0, vals)
            inv = 1.0 / (tsum + 1e-8)
            w_refs[r][...] = (wvec * inv).reshape(1, 1, k)
            i_refs[r][...] = ivec.reshape(1, 1, k)
            m_refs[r][...] = mask.reshape(1, 1, _N)


def kernel(x, importance, Wc, WQ, WK, WV):
    w_all = jnp.concatenate([Wc, WQ, WK, WV], axis=1)
    imp4 = importance.reshape(_B, _NSB, _SBLK, 1)

    def out(shape, dtype=jnp.float32):
        return jax.ShapeDtypeStruct(shape, dtype)

    out_shapes = (
        out((_B, 1, _KS[0])), out((_B, 1, _KS[0]), jnp.int32),
        out((_B, 1, _KS[1])), out((_B, 1, _KS[1]), jnp.int32),
        out((_B, 1, _KS[2])), out((_B, 1, _KS[2]), jnp.int32),
        out((_B, 1, _KS[3])), out((_B, 1, _KS[3]), jnp.int32),
        out((_B, 1, _N)), out((_B, 1, _N)), out((_B, 1, _N)), out((_B, 1, _N)),
    )

    def b_map(b, s):
        return (b, 0, 0)

    out_specs = tuple(
        pl.BlockSpec((1, 1, sh.shape[2]), b_map) for sh in out_shapes)

    res = pl.pallas_call(
        _router_kernel,
        grid=(_B, _NSB),
        in_specs=[
            pl.BlockSpec((1, _SBLK, _D), lambda b, s: (b, s, 0)),
            pl.BlockSpec((1, 1, _SBLK, 1), lambda b, s: (b, s, 0, 0)),
            pl.BlockSpec((_D, _NR * _N), lambda b, s: (0, 0)),
        ],
        out_specs=out_specs,
        out_shape=out_shapes,
        scratch_shapes=[pltpu.VMEM((_NR * _N, 1), jnp.float32)],
        compiler_params=pltpu.CompilerParams(
            dimension_semantics=("arbitrary", "arbitrary")),
    )(x, imp4, w_all)

    (cw, ci, qw, qi, kw, ki, vw, vi, cm, qm, km, vm) = res
    sq = lambda a: a.reshape(a.shape[0], a.shape[2])
    return (sq(cw), sq(ci), sq(qw), sq(qi), sq(kw), sq(ki), sq(vw), sq(vi),
            sq(cm), sq(qm), sq(km), sq(vm))


# speedup vs baseline: 2.2212x; 1.1183x over previous
import jax
import jax.numpy as jnp
from jax.experimental import pallas as pl
from jax.experimental.pallas import tpu as pltpu

_B, _S, _D = 4, 2048, 2048
_N = 64
_NR = 4
_KS = (16, 8, 8, 8)
_SBLK = 1024
_NSB = _S // _SBLK


def _router_kernel(x_ref, imp_ref, w_ref,
                   cw_ref, ci_ref, qw_ref, qi_ref, kw_ref, ki_ref,
                   vw_ref, vi_ref, cm_ref, qm_ref, km_ref, vm_ref,
                   acc_ref):
    s = pl.program_id(1)

    @pl.when(s == 0)
    def _zero():
        acc_ref[...] = jnp.zeros_like(acc_ref)

    x = x_ref[0].astype(jnp.bfloat16)
    imp = imp_ref[0, 0].astype(jnp.bfloat16)
    logits = jax.lax.dot_general(
        x, w_ref[...].astype(jnp.bfloat16), (((1,), (0,)), ((), ())),
        preferred_element_type=jnp.float32)
    lt = logits.T

    for r in range(_NR):
        lg = lt[r * _N:(r + 1) * _N, :]
        m = jnp.max(lg, axis=0, keepdims=True)
        e = jnp.exp(lg - m)
        esum = jnp.sum(e, axis=0, keepdims=True)
        p16 = (e / esum).astype(jnp.bfloat16)
        contrib = jax.lax.dot_general(
            p16, imp, (((1,), (0,)), ((), ())),
            preferred_element_type=jnp.float32)
        acc_ref[r * _N:(r + 1) * _N, :] += contrib

    @pl.when(s == _NSB - 1)
    def _epilogue():
        iota = jax.lax.broadcasted_iota(jnp.int32, (1, _N), 1)
        w_refs = (cw_ref, qw_ref, kw_ref, vw_ref)
        i_refs = (ci_ref, qi_ref, ki_ref, vi_ref)
        m_refs = (cm_ref, qm_ref, km_ref, vm_ref)
        for r in range(_NR):
            k = _KS[r]
            kiota = jax.lax.broadcasted_iota(jnp.int32, (1, k), 1)
            w = acc_ref[r * _N:(r + 1) * _N, :].T
            wn = w / (jnp.sum(w, axis=-1, keepdims=True) + 1e-8)
            vals = wn
            mask = jnp.zeros((1, _N), jnp.float32)
            wvec = jnp.zeros((1, k), jnp.float32)
            ivec = jnp.zeros((1, k), jnp.int32)
            tsum = jnp.zeros((1, 1), jnp.float32)
            for j in range(k):
                mx = jnp.max(vals, axis=-1, keepdims=True)
                ixv = jnp.min(jnp.where(vals == mx, iota, _N),
                              axis=-1, keepdims=True)
                tsum = tsum + mx
                wvec = jnp.where(kiota == j, mx, wvec)
                ivec = jnp.where(kiota == j, ixv, ivec)
                hit = iota == ixv
                mask = jnp.where(hit, 1.0, mask)
                vals = jnp.where(hit, -1.0, vals)
            inv = 1.0 / (tsum + 1e-8)
            w_refs[r][...] = (wvec * inv).reshape(1, 1, k)
            i_refs[r][...] = ivec.reshape(1, 1, k)
            m_refs[r][...] = mask.reshape(1, 1, _N)


def kernel(x, importance, Wc, WQ, WK, WV):
    w_all = jnp.concatenate([Wc, WQ, WK, WV], axis=1)
    imp4 = importance.reshape(_B, _NSB, _SBLK, 1)

    def out(shape, dtype=jnp.float32):
        return jax.ShapeDtypeStruct(shape, dtype)

    out_shapes = (
        out((_B, 1, _KS[0])), out((_B, 1, _KS[0]), jnp.int32),
        out((_B, 1, _KS[1])), out((_B, 1, _KS[1]), jnp.int32),
        out((_B, 1, _KS[2])), out((_B, 1, _KS[2]), jnp.int32),
        out((_B, 1, _KS[3])), out((_B, 1, _KS[3]), jnp.int32),
        out((_B, 1, _N)), out((_B, 1, _N)), out((_B, 1, _N)), out((_B, 1, _N)),
    )

    def b_map(b, s):
        return (b, 0, 0)

    out_specs = tuple(
        pl.BlockSpec((1, 1, sh.shape[2]), b_map) for sh in out_shapes)

    res = pl.pallas_call(
        _router_kernel,
        grid=(_B, _NSB),
        in_specs=[
            pl.BlockSpec((1, _SBLK, _D), lambda b, s: (b, s, 0)),
            pl.BlockSpec((1, 1, _SBLK, 1), lambda b, s: (b, s, 0, 0)),
            pl.BlockSpec((_D, _NR * _N), lambda b, s: (0, 0)),
        ],
        out_specs=out_specs,
        out_shape=out_shapes,
        scratch_shapes=[pltpu.VMEM((_NR * _N, 1), jnp.float32)],
        compiler_params=pltpu.CompilerParams(
            dimension_semantics=("arbitrary", "arbitrary")),
    )(x, imp4, w_all)

    (cw, ci, qw, qi, kw, ki, vw, vi, cm, qm, km, vm) = res
    sq = lambda a: a.reshape(a.shape[0], a.shape[2])
    return (sq(cw), sq(ci), sq(qw), sq(qi), sq(kw), sq(ki), sq(vw), sq(vi),
            sq(cm), sq(qm), sq(km), sq(vm))


# speedup vs baseline: 2.6461x; 1.1913x over previous
import jax
import jax.numpy as jnp
from jax.experimental import pallas as pl
from jax.experimental.pallas import tpu as pltpu

_B, _S, _D = 4, 2048, 2048
_N = 64
_NR = 4
_KS = (16, 8, 8, 8)
_SBLK = 2048
_NSB = _S // _SBLK


def _router_kernel(x_ref, imp_ref, w_ref,
                   cw_ref, ci_ref, qw_ref, qi_ref, kw_ref, ki_ref,
                   vw_ref, vi_ref, cm_ref, qm_ref, km_ref, vm_ref,
                   acc_ref):
    s = pl.program_id(1)

    @pl.when(s == 0)
    def _zero():
        acc_ref[...] = jnp.zeros_like(acc_ref)

    x = x_ref[0].astype(jnp.bfloat16)
    imp = imp_ref[0, 0].astype(jnp.bfloat16)
    logits = jax.lax.dot_general(
        x, w_ref[...].astype(jnp.bfloat16), (((1,), (0,)), ((), ())),
        preferred_element_type=jnp.float32)
    lt = logits.T

    for r in range(_NR):
        lg = lt[r * _N:(r + 1) * _N, :]
        m = jnp.max(lg, axis=0, keepdims=True)
        e = jnp.exp(lg - m)
        esum = jnp.sum(e, axis=0, keepdims=True)
        p16 = (e / esum).astype(jnp.bfloat16)
        contrib = jax.lax.dot_general(
            p16, imp, (((1,), (0,)), ((), ())),
            preferred_element_type=jnp.float32)
        acc_ref[r * _N:(r + 1) * _N, :] += contrib

    @pl.when(s == _NSB - 1)
    def _epilogue():
        iota = jax.lax.broadcasted_iota(jnp.int32, (1, _N), 1)
        w_refs = (cw_ref, qw_ref, kw_ref, vw_ref)
        i_refs = (ci_ref, qi_ref, ki_ref, vi_ref)
        m_refs = (cm_ref, qm_ref, km_ref, vm_ref)
        for r in range(_NR):
            k = _KS[r]
            kiota = jax.lax.broadcasted_iota(jnp.int32, (1, k), 1)
            w = acc_ref[r * _N:(r + 1) * _N, :].T
            wn = w / (jnp.sum(w, axis=-1, keepdims=True) + 1e-8)
            vals = wn
            mask = jnp.zeros((1, _N), jnp.float32)
            wvec = jnp.zeros((1, k), jnp.float32)
            ivec = jnp.zeros((1, k), jnp.int32)
            tsum = jnp.zeros((1, 1), jnp.float32)
            for j in range(k):
                mx = jnp.max(vals, axis=-1, keepdims=True)
                ixv = jnp.min(jnp.where(vals == mx, iota, _N),
                              axis=-1, keepdims=True)
                tsum = tsum + mx
                wvec = jnp.where(kiota == j, mx, wvec)
                ivec = jnp.where(kiota == j, ixv, ivec)
                hit = iota == ixv
                mask = jnp.where(hit, 1.0, mask)
                vals = jnp.where(hit, -1.0, vals)
            inv = 1.0 / (tsum + 1e-8)
            w_refs[r][...] = (wvec * inv).reshape(1, 1, k)
            i_refs[r][...] = ivec.reshape(1, 1, k)
            m_refs[r][...] = mask.reshape(1, 1, _N)


def kernel(x, importance, Wc, WQ, WK, WV):
    w_all = jnp.concatenate([Wc, WQ, WK, WV], axis=1)
    imp4 = importance.reshape(_B, _NSB, _SBLK, 1)

    def out(shape, dtype=jnp.float32):
        return jax.ShapeDtypeStruct(shape, dtype)

    out_shapes = (
        out((_B, 1, _KS[0])), out((_B, 1, _KS[0]), jnp.int32),
        out((_B, 1, _KS[1])), out((_B, 1, _KS[1]), jnp.int32),
        out((_B, 1, _KS[2])), out((_B, 1, _KS[2]), jnp.int32),
        out((_B, 1, _KS[3])), out((_B, 1, _KS[3]), jnp.int32),
        out((_B, 1, _N)), out((_B, 1, _N)), out((_B, 1, _N)), out((_B, 1, _N)),
    )

    def b_map(b, s):
        return (b, 0, 0)

    out_specs = tuple(
        pl.BlockSpec((1, 1, sh.shape[2]), b_map) for sh in out_shapes)

    res = pl.pallas_call(
        _router_kernel,
        grid=(_B, _NSB),
        in_specs=[
            pl.BlockSpec((1, _SBLK, _D), lambda b, s: (b, s, 0)),
            pl.BlockSpec((1, 1, _SBLK, 1), lambda b, s: (b, s, 0, 0)),
            pl.BlockSpec((_D, _NR * _N), lambda b, s: (0, 0)),
        ],
        out_specs=out_specs,
        out_shape=out_shapes,
        scratch_shapes=[pltpu.VMEM((_NR * _N, 1), jnp.float32)],
        compiler_params=pltpu.CompilerParams(
            dimension_semantics=("arbitrary", "arbitrary")),
    )(x, imp4, w_all)

    (cw, ci, qw, qi, kw, ki, vw, vi, cm, qm, km, vm) = res
    sq = lambda a: a.reshape(a.shape[0], a.shape[2])
    return (sq(cw), sq(ci), sq(qw), sq(qi), sq(kw), sq(ki), sq(vw), sq(vi),
            sq(cm), sq(qm), sq(km), sq(vm))


# speedup vs baseline: 2.7262x; 1.0303x over previous
import jax
import jax.numpy as jnp
from jax.experimental import pallas as pl
from jax.experimental.pallas import tpu as pltpu

_B, _S, _D = 4, 2048, 2048
_N = 64
_NR = 4
_KS = (16, 8, 8, 8)
_SBLK = 2048
_NSB = _S // _SBLK
_NCH = 2


def _router_kernel(x_ref, imp_ref, w_ref,
                   cw_ref, ci_ref, qw_ref, qi_ref, kw_ref, ki_ref,
                   vw_ref, vi_ref, cm_ref, qm_ref, km_ref, vm_ref,
                   acc_ref):
    s = pl.program_id(1)

    @pl.when(s == 0)
    def _zero():
        acc_ref[...] = jnp.zeros_like(acc_ref)

    w16 = w_ref[...].astype(jnp.bfloat16)
    csz = _SBLK // _NCH
    contribs = [None] * _NR
    for c in range(_NCH):
        x = x_ref[0][c * csz:(c + 1) * csz, :].astype(jnp.bfloat16)
        imp = imp_ref[0, 0][c * csz:(c + 1) * csz, :].astype(jnp.bfloat16)
        logits = jax.lax.dot_general(
            x, w16, (((1,), (0,)), ((), ())),
            preferred_element_type=jnp.float32)
        lt = logits.T
        for r in range(_NR):
            lg = lt[r * _N:(r + 1) * _N, :]
            m = jnp.max(lg, axis=0, keepdims=True)
            e = jnp.exp(lg - m)
            esum = jnp.sum(e, axis=0, keepdims=True)
            p16 = (e / esum).astype(jnp.bfloat16)
            contrib = jax.lax.dot_general(
                p16, imp, (((1,), (0,)), ((), ())),
                preferred_element_type=jnp.float32)
            contribs[r] = contrib if contribs[r] is None else contribs[r] + contrib
    for r in range(_NR):
        acc_ref[r * _N:(r + 1) * _N, :] += contribs[r]

    @pl.when(s == _NSB - 1)
    def _epilogue():
        iota = jax.lax.broadcasted_iota(jnp.int32, (1, _N), 1)
        w_refs = (cw_ref, qw_ref, kw_ref, vw_ref)
        i_refs = (ci_ref, qi_ref, ki_ref, vi_ref)
        m_refs = (cm_ref, qm_ref, km_ref, vm_ref)
        for r in range(_NR):
            k = _KS[r]
            kiota = jax.lax.broadcasted_iota(jnp.int32, (1, k), 1)
            w = acc_ref[r * _N:(r + 1) * _N, :].T
            wn = w / (jnp.sum(w, axis=-1, keepdims=True) + 1e-8)
            vals = wn
            mask = jnp.zeros((1, _N), jnp.float32)
            wvec = jnp.zeros((1, k), jnp.float32)
            ivec = jnp.zeros((1, k), jnp.int32)
            tsum = jnp.zeros((1, 1), jnp.float32)
            for j in range(k):
                mx = jnp.max(vals, axis=-1, keepdims=True)
                ixv = jnp.min(jnp.where(vals == mx, iota, _N),
                              axis=-1, keepdims=True)
                tsum = tsum + mx
                wvec = jnp.where(kiota == j, mx, wvec)
                ivec = jnp.where(kiota == j, ixv, ivec)
                hit = iota == ixv
                mask = jnp.where(hit, 1.0, mask)
                vals = jnp.where(hit, -1.0, vals)
            inv = 1.0 / (tsum + 1e-8)
            w_refs[r][...] = (wvec * inv).reshape(1, 1, k)
            i_refs[r][...] = ivec.reshape(1, 1, k)
            m_refs[r][...] = mask.reshape(1, 1, _N)


def kernel(x, importance, Wc, WQ, WK, WV):
    w_all = jnp.concatenate([Wc, WQ, WK, WV], axis=1)
    imp4 = importance.reshape(_B, _NSB, _SBLK, 1)

    def out(shape, dtype=jnp.float32):
        return jax.ShapeDtypeStruct(shape, dtype)

    out_shapes = (
        out((_B, 1, _KS[0])), out((_B, 1, _KS[0]), jnp.int32),
        out((_B, 1, _KS[1])), out((_B, 1, _KS[1]), jnp.int32),
        out((_B, 1, _KS[2])), out((_B, 1, _KS[2]), jnp.int32),
        out((_B, 1, _KS[3])), out((_B, 1, _KS[3]), jnp.int32),
        out((_B, 1, _N)), out((_B, 1, _N)), out((_B, 1, _N)), out((_B, 1, _N)),
    )

    def b_map(b, s):
        return (b, 0, 0)

    out_specs = tuple(
        pl.BlockSpec((1, 1, sh.shape[2]), b_map) for sh in out_shapes)

    res = pl.pallas_call(
        _router_kernel,
        grid=(_B, _NSB),
        in_specs=[
            pl.BlockSpec((1, _SBLK, _D), lambda b, s: (b, s, 0)),
            pl.BlockSpec((1, 1, _SBLK, 1), lambda b, s: (b, s, 0, 0)),
            pl.BlockSpec((_D, _NR * _N), lambda b, s: (0, 0)),
        ],
        out_specs=out_specs,
        out_shape=out_shapes,
        scratch_shapes=[pltpu.VMEM((_NR * _N, 1), jnp.float32)],
        compiler_params=pltpu.CompilerParams(
            dimension_semantics=("arbitrary", "arbitrary")),
    )(x, imp4, w_all)

    (cw, ci, qw, qi, kw, ki, vw, vi, cm, qm, km, vm) = res
    sq = lambda a: a.reshape(a.shape[0], a.shape[2])
    return (sq(cw), sq(ci), sq(qw), sq(qi), sq(kw), sq(ki), sq(vw), sq(vi),
            sq(cm), sq(qm), sq(km), sq(vm))
